# Initial kernel scaffold; baseline (speedup 1.0000x reference)
import jax
import jax.numpy as jnp
from jax.experimental import pallas as pl


def kernel(x, pos, edge_index, batch, W_emb, b_emb, eps, Wc1, bc1, g1, be1, m1, v1, Wc2, bc2, g2, be2, m2, v2, W1, b1, W2, b2):
    raise NotImplementedError("write your pallas kernel here")


# speedup vs baseline: 4.0018x; 4.0018x over previous
import functools

import jax
import jax.numpy as jnp
from jax import lax
from jax.experimental import pallas as pl
from jax.experimental.pallas import tpu as pltpu
from jax.experimental.pallas import tpu_sc as plsc

N = 10000
H = 128
G = 64
L = 3
OUT = 10

TILES = 32
LANE = 128
NPAD = 10112
ROWS_PER_TILE = NPAD // 16

ROW_BLK = 1000
POOL_BLK = 400
PRECISION = lax.Precision.HIGHEST


def _sc_aggregate_body(h_hbm, srcg_hbm, dstg_hbm, zeros_hbm, part_hbm,
                       src_v, dst_v, rows_v, shared, sem, groups):
    c = lax.axis_index("c")
    s = lax.axis_index("s")
    tile = c * 16 + s

    pltpu.sync_copy(zeros_hbm, shared.at[pl.ds(s * ROWS_PER_TILE, ROWS_PER_TILE)])
    plsc.subcore_barrier()

    pltpu.sync_copy(srcg_hbm.at[tile], src_v)
    pltpu.sync_copy(dstg_hbm.at[tile], dst_v)

    def body(j, carry):
        pltpu.async_copy(h_hbm.at[src_v.at[j]], rows_v, sem).wait()
        pltpu.sync_copy(rows_v, shared.at[dst_v.at[j]], add=True)
        return carry

    lax.fori_loop(0, groups, body, 0)
    plsc.subcore_barrier()

    pltpu.sync_copy(shared.at[pl.ds(s * ROWS_PER_TILE, ROWS_PER_TILE)],
                    part_hbm.at[c, pl.ds(s * ROWS_PER_TILE, ROWS_PER_TILE)])


def _make_sc_aggregate(groups):
    mesh = plsc.VectorSubcoreMesh(core_axis_name="c", subcore_axis_name="s")
    return pl.kernel(
        functools.partial(_sc_aggregate_body, groups=groups),
        out_type=jax.ShapeDtypeStruct((2, NPAD, H), jnp.float32),
        mesh=mesh,
        scratch_types=[
            pltpu.VMEM((groups, LANE), jnp.int32),
            pltpu.VMEM((groups, LANE), jnp.int32),
            pltpu.VMEM((LANE, H), jnp.float32),
            pltpu.VMEM_SHARED((NPAD, H), jnp.float32),
            pltpu.SemaphoreType.DMA,
        ],
    )


def _embed_body(x_ref, p_ref, wx_ref, wp_ref, b_ref, o_ref):
    acc = jnp.dot(x_ref[...], wx_ref[...], precision=PRECISION)
    acc += jnp.dot(p_ref[...], wp_ref[...], precision=PRECISION)
    o_ref[...] = acc + b_ref[...]


def _embed(x, pos_p, wxt, wpt, brow):
    grid = N // ROW_BLK
    return pl.pallas_call(
        _embed_body,
        grid=(grid,),
        in_specs=[
            pl.BlockSpec((ROW_BLK, H), lambda i: (i, 0)),
            pl.BlockSpec((ROW_BLK, H), lambda i: (i, 0)),
            pl.BlockSpec((H, H), lambda i: (0, 0)),
            pl.BlockSpec((H, H), lambda i: (0, 0)),
            pl.BlockSpec((1, H), lambda i: (0, 0)),
        ],
        out_specs=pl.BlockSpec((ROW_BLK, H), lambda i: (i, 0)),
        out_shape=jax.ShapeDtypeStruct((N, H), jnp.float32),
    )(x, pos_p, wxt, wpt, brow)


def _mlp_body(h_ref, part_ref, sc_ref, w1_ref, b1_ref, w2_ref, b2_ref, o_ref):
    hin = h_ref[...] * sc_ref[...] + part_ref[0] + part_ref[1]
    y = jnp.dot(hin, w1_ref[...], precision=PRECISION) + b1_ref[...]
    y = jnp.maximum(y, 0.0)
    z = jnp.dot(y, w2_ref[...], precision=PRECISION) + b2_ref[...]
    o_ref[...] = jnp.maximum(z, 0.0)


def _mlp(h, part, scale_row, w1t, b1row, w2t, b2row):
    grid = N // ROW_BLK
    return pl.pallas_call(
        _mlp_body,
        grid=(grid,),
        in_specs=[
            pl.BlockSpec((ROW_BLK, H), lambda i: (i, 0)),
            pl.BlockSpec((2, ROW_BLK, H), lambda i: (0, i, 0)),
            pl.BlockSpec((1, H), lambda i: (0, 0)),
            pl.BlockSpec((H, H), lambda i: (0, 0)),
            pl.BlockSpec((1, H), lambda i: (0, 0)),
            pl.BlockSpec((H, H), lambda i: (0, 0)),
            pl.BlockSpec((1, H), lambda i: (0, 0)),
        ],
        out_specs=pl.BlockSpec((ROW_BLK, H), lambda i: (i, 0)),
        out_shape=jax.ShapeDtypeStruct((N, H), jnp.float32),
    )(h, part, scale_row, w1t, b1row, w2t, b2row)


def _pool_body(h_ref, b3_ref, w1_ref, b1_ref, w2_ref, b2_ref, o_ref, acc_ref):
    step = pl.program_id(0)

    @pl.when(step == 0)
    def _():
        acc_ref[...] = jnp.full((G, H), -jnp.inf, jnp.float32)

    bb = b3_ref[0]
    hb = h_ref[...]
    for g in range(G):
        m = jnp.max(jnp.where(bb == g, hb, -jnp.inf), axis=0)
        acc_ref[g, :] = jnp.maximum(acc_ref[g, :], m)

    @pl.when(step == pl.num_programs(0) - 1)
    def _():
        pooled = acc_ref[...]
        y = jnp.dot(pooled, w1_ref[...], precision=PRECISION) + b1_ref[...]
        y = jnp.maximum(y, 0.0)
        o_ref[...] = jnp.dot(y, w2_ref[...], precision=PRECISION) + b2_ref[...]


def _pool_head(h, batch3, w1t, b1row, w2tp, b2row):
    grid = N // POOL_BLK
    return pl.pallas_call(
        _pool_body,
        grid=(grid,),
        in_specs=[
            pl.BlockSpec((POOL_BLK, H), lambda i: (i, 0)),
            pl.BlockSpec((1, POOL_BLK, 1), lambda i: (i, 0, 0)),
            pl.BlockSpec((H, H), lambda i: (0, 0)),
            pl.BlockSpec((1, H), lambda i: (0, 0)),
            pl.BlockSpec((H, H), lambda i: (0, 0)),
            pl.BlockSpec((1, H), lambda i: (0, 0)),
        ],
        out_specs=pl.BlockSpec((G, H), lambda i: (0, 0)),
        out_shape=jax.ShapeDtypeStruct((G, H), jnp.float32),
        scratch_shapes=[pltpu.VMEM((G, H), jnp.float32)],
    )(h, batch3, w1t, b1row, w2tp, b2row)


def kernel(x, pos, edge_index, batch, W_emb, b_emb, eps, Wc1, bc1, g1, be1,
           m1, v1, Wc2, bc2, g2, be2, m2, v2, W1, b1, W2, b2):
    E = edge_index.shape[1]
    groups = -(-E // (TILES * LANE))
    e_pad = TILES * groups * LANE

    src = edge_index[0]
    dst = edge_index[1]
    srcg = jnp.concatenate(
        [src, jnp.zeros((e_pad - E,), jnp.int32)]).reshape(TILES, groups, LANE)
    dstg = jnp.concatenate(
        [dst, jnp.full((e_pad - E,), N, jnp.int32)]).reshape(TILES, groups, LANE)
    zeros_tile = jnp.zeros((ROWS_PER_TILE, H), jnp.float32)

    pos_p = jnp.pad(pos, ((0, 0), (0, H - pos.shape[1])))
    wxt = W_emb[:, :H].T
    wpt = jnp.pad(W_emb[:, H:].T, ((0, H - (W_emb.shape[1] - H)), (0, 0)))
    h = _embed(x, pos_p, wxt, wpt, b_emb.reshape(1, H))

    s1 = g1 / jnp.sqrt(v1 + 1e-5)
    s2 = g2 / jnp.sqrt(v2 + 1e-5)
    w1f = Wc1 * s1[:, :, None]
    b1f = (bc1 - m1) * s1 + be1
    w2f = Wc2 * s2[:, :, None]
    b2f = (bc2 - m2) * s2 + be2

    sc_aggregate = _make_sc_aggregate(groups)
    for i in range(L):
        part = sc_aggregate(h, srcg, dstg, zeros_tile)
        h = _mlp(h, part,
                 jnp.full((1, H), 1.0 + eps[i], jnp.float32),
                 w1f[i].T, b1f[i].reshape(1, H),
                 w2f[i].T, b2f[i].reshape(1, H))

    batch3 = batch.reshape(N // POOL_BLK, POOL_BLK, 1)
    w2tp = jnp.pad(W2.T, ((0, 0), (0, H - OUT)))
    b2p = jnp.pad(b2, (0, H - OUT)).reshape(1, H)
    out = _pool_head(h, batch3, W1.T, b1.reshape(1, H), w2tp, b2p)
    return out[:, :OUT]
